# baseline (speedup 1.0000x reference)
import jax
import jax.numpy as jnp
from jax.experimental import pallas as pl
from jax.experimental.pallas import tpu as pltpu

_NUM_EXPERTS = 8
_D = 768
_HEADS = 12
_DEPTH = 64
_LANE = 128
_BT = 1024
_NEG = -1e30


def _fused_body(x_ref, wr_ref, br_ref, we_ref, be_ref, wq_ref, bq_ref,
                wk_ref, bk_ref, wv_ref, bv_ref, wo_ref, bo_ref, o_ref,
                wrp_ref, brp_ref, hmap_ref, hmapt_ref, wqkv_ref, bqkv_ref):
    i = pl.program_id(0)

    @pl.when(i == 0)
    def _prep():
        wrp_ref[...] = jnp.concatenate(
            [wr_ref[...], jnp.zeros((_D, _LANE - _NUM_EXPERTS), jnp.float32)],
            axis=1)
        brp_ref[...] = jnp.concatenate(
            [br_ref[...], jnp.full((1, _LANE - _NUM_EXPERTS), _NEG,
                                   jnp.float32)], axis=1)
        di = jax.lax.broadcasted_iota(jnp.int32, (_D, _LANE), 0) // _DEPTH
        hi = jax.lax.broadcasted_iota(jnp.int32, (_D, _LANE), 1)
        hmap_ref[...] = (di == hi).astype(jnp.float32)
        dit = jax.lax.broadcasted_iota(jnp.int32, (_LANE, _D), 1) // _DEPTH
        hit = jax.lax.broadcasted_iota(jnp.int32, (_LANE, _D), 0)
        hmapt_ref[...] = (dit == hit).astype(jnp.float32)
        wqkv_ref[:, 0:_D] = wq_ref[...]
        wqkv_ref[:, _D:2 * _D] = wk_ref[...]
        wqkv_ref[:, 2 * _D:3 * _D] = wv_ref[...]
        bqkv_ref[:, 0:_D] = bq_ref[...]
        bqkv_ref[:, _D:2 * _D] = bk_ref[...]
        bqkv_ref[:, 2 * _D:3 * _D] = bv_ref[...]

    x = x_ref[...]

    logits = jnp.dot(x, wrp_ref[...], preferred_element_type=jnp.float32)
    logits = logits + brp_ref[...]
    m = jnp.max(logits, axis=-1, keepdims=True)
    e = jnp.exp(logits - m)
    probs = e / jnp.sum(e, axis=-1, keepdims=True)

    cols = jax.lax.broadcasted_iota(jnp.int32, probs.shape, 1)
    p1 = jnp.max(probs, axis=-1, keepdims=True)
    i1 = jnp.min(jnp.where(probs >= p1, cols, _LANE), axis=-1, keepdims=True)
    probs_m = jnp.where(cols == i1, -1.0, probs)
    p2 = jnp.max(probs_m, axis=-1, keepdims=True)
    i2 = jnp.min(jnp.where(probs_m >= p2, cols, _LANE), axis=-1, keepdims=True)
    sel = (cols == i1) | (cols == i2)
    w = jnp.where(sel, probs, 0.0)

    combined = jnp.zeros((x.shape[0], _D), dtype=jnp.float32)
    for i_e in range(_NUM_EXPERTS):
        eo = jnp.dot(x, we_ref[i_e], preferred_element_type=jnp.float32)
        eo = jnp.maximum(eo + be_ref[i_e:i_e + 1, :], 0.0)
        combined = combined + eo * w[:, i_e:i_e + 1]

    qkv = jnp.dot(combined, wqkv_ref[...],
                  preferred_element_type=jnp.float32) + bqkv_ref[...]
    q = qkv[:, 0:_D]
    k = qkv[:, _D:2 * _D]
    v = qkv[:, 2 * _D:3 * _D]
    s = jnp.dot(q * k, hmap_ref[...], preferred_element_type=jnp.float32)
    s = s * (1.0 / jnp.sqrt(jnp.float32(_DEPTH)))
    s = jnp.where(cols < _HEADS, s, _NEG)
    sm = jnp.max(s, axis=-1, keepdims=True)
    se = jnp.exp(s - sm)
    aw = se / jnp.sum(se, axis=-1, keepdims=True)
    wb = jnp.dot(aw, hmapt_ref[...], preferred_element_type=jnp.float32)
    out = jnp.dot(wb * v, wo_ref[...], preferred_element_type=jnp.float32)
    o_ref[...] = out + bo_ref[...]


@jax.jit
def kernel(inputs, Wr, br, We, be, Wq, bq, Wk, bk, Wv, bv, Wo, bo):
    n = inputs.shape[0]
    grid = (n // _BT,)
    full = lambda shape: pl.BlockSpec(shape, lambda i: (0,) * len(shape))
    out = pl.pallas_call(
        _fused_body,
        grid=grid,
        in_specs=[
            pl.BlockSpec((_BT, _D), lambda i: (i, 0)),
            full((_D, _NUM_EXPERTS)),
            full((1, _NUM_EXPERTS)),
            full((_NUM_EXPERTS, _D, _D)),
            full((_NUM_EXPERTS, _D)),
            full((_D, _D)), full((1, _D)),
            full((_D, _D)), full((1, _D)),
            full((_D, _D)), full((1, _D)),
            full((_D, _D)), full((1, _D)),
        ],
        out_specs=pl.BlockSpec((_BT, _D), lambda i: (i, 0)),
        out_shape=jax.ShapeDtypeStruct((n, _D), jnp.float32),
        scratch_shapes=[
            pltpu.VMEM((_D, _LANE), jnp.float32),
            pltpu.VMEM((1, _LANE), jnp.float32),
            pltpu.VMEM((_D, _LANE), jnp.float32),
            pltpu.VMEM((_LANE, _D), jnp.float32),
            pltpu.VMEM((_D, 3 * _D), jnp.float32),
            pltpu.VMEM((1, 3 * _D), jnp.float32),
        ],
        compiler_params=pltpu.CompilerParams(
            dimension_semantics=("arbitrary",),
        ),
    )(inputs, Wr, br.reshape(1, _NUM_EXPERTS), We, be,
      Wq, bq.reshape(1, _D), Wk, bk.reshape(1, _D),
      Wv, bv.reshape(1, _D), Wo, bo.reshape(1, _D))
    return out


# speedup vs baseline: 1.0055x; 1.0055x over previous
import jax
import jax.numpy as jnp
from jax.experimental import pallas as pl
from jax.experimental.pallas import tpu as pltpu

_NUM_EXPERTS = 8
_D = 768
_HEADS = 12
_DEPTH = 64
_LANE = 128
_BT = 1024
_NEG = -1e30


def _fused_body(x_ref, wr_ref, br_ref, we_ref, be_ref, wq_ref, bq_ref,
                wk_ref, bk_ref, wv_ref, bv_ref, wo_ref, bo_ref, o_ref,
                wrp_ref, brp_ref, hmap_ref, hmapt_ref):
    if True:
        wrp_ref[...] = jnp.concatenate(
            [wr_ref[...], jnp.zeros((_D, _LANE - _NUM_EXPERTS), jnp.float32)],
            axis=1)
        brp_ref[...] = jnp.concatenate(
            [br_ref[...], jnp.full((1, _LANE - _NUM_EXPERTS), _NEG,
                                   jnp.float32)], axis=1)
        di = jax.lax.broadcasted_iota(jnp.int32, (_D, _LANE), 0) // _DEPTH
        hi = jax.lax.broadcasted_iota(jnp.int32, (_D, _LANE), 1)
        hmap_ref[...] = (di == hi).astype(jnp.float32)
        dit = jax.lax.broadcasted_iota(jnp.int32, (_LANE, _D), 1) // _DEPTH
        hit = jax.lax.broadcasted_iota(jnp.int32, (_LANE, _D), 0)
        hmapt_ref[...] = (dit == hit).astype(jnp.float32)

    x = x_ref[...]

    logits = jnp.dot(x, wrp_ref[...], preferred_element_type=jnp.float32)
    logits = logits + brp_ref[...]
    m = jnp.max(logits, axis=-1, keepdims=True)
    e = jnp.exp(logits - m)
    probs = e / jnp.sum(e, axis=-1, keepdims=True)

    cols = jax.lax.broadcasted_iota(jnp.int32, probs.shape, 1)
    p1 = jnp.max(probs, axis=-1, keepdims=True)
    i1 = jnp.min(jnp.where(probs >= p1, cols, _LANE), axis=-1, keepdims=True)
    probs_m = jnp.where(cols == i1, -1.0, probs)
    p2 = jnp.max(probs_m, axis=-1, keepdims=True)
    i2 = jnp.min(jnp.where(probs_m >= p2, cols, _LANE), axis=-1, keepdims=True)
    sel = (cols == i1) | (cols == i2)
    w = jnp.where(sel, probs, 0.0)

    combined = jnp.zeros((x.shape[0], _D), dtype=jnp.float32)
    for i_e in range(_NUM_EXPERTS):
        eo = jnp.dot(x, we_ref[i_e], preferred_element_type=jnp.float32)
        eo = jnp.maximum(eo + be_ref[i_e:i_e + 1, :], 0.0)
        combined = combined + eo * w[:, i_e:i_e + 1]

    q = jnp.dot(combined, wq_ref[...], preferred_element_type=jnp.float32) + bq_ref[...]
    k = jnp.dot(combined, wk_ref[...], preferred_element_type=jnp.float32) + bk_ref[...]
    v = jnp.dot(combined, wv_ref[...], preferred_element_type=jnp.float32) + bv_ref[...]
    s = jnp.dot(q * k, hmap_ref[...], preferred_element_type=jnp.float32)
    s = s * (1.0 / jnp.sqrt(jnp.float32(_DEPTH)))
    s = jnp.where(cols < _HEADS, s, _NEG)
    sm = jnp.max(s, axis=-1, keepdims=True)
    se = jnp.exp(s - sm)
    aw = se / jnp.sum(se, axis=-1, keepdims=True)
    wb = jnp.dot(aw, hmapt_ref[...], preferred_element_type=jnp.float32)
    out = jnp.dot(wb * v, wo_ref[...], preferred_element_type=jnp.float32)
    o_ref[...] = out + bo_ref[...]


@jax.jit
def kernel(inputs, Wr, br, We, be, Wq, bq, Wk, bk, Wv, bv, Wo, bo):
    n = inputs.shape[0]
    grid = (n // _BT,)
    full = lambda shape: pl.BlockSpec(shape, lambda i: (0,) * len(shape))
    out = pl.pallas_call(
        _fused_body,
        grid=grid,
        in_specs=[
            pl.BlockSpec((_BT, _D), lambda i: (i, 0)),
            full((_D, _NUM_EXPERTS)),
            full((1, _NUM_EXPERTS)),
            full((_NUM_EXPERTS, _D, _D)),
            full((_NUM_EXPERTS, _D)),
            full((_D, _D)), full((1, _D)),
            full((_D, _D)), full((1, _D)),
            full((_D, _D)), full((1, _D)),
            full((_D, _D)), full((1, _D)),
        ],
        out_specs=pl.BlockSpec((_BT, _D), lambda i: (i, 0)),
        out_shape=jax.ShapeDtypeStruct((n, _D), jnp.float32),
        scratch_shapes=[
            pltpu.VMEM((_D, _LANE), jnp.float32),
            pltpu.VMEM((1, _LANE), jnp.float32),
            pltpu.VMEM((_D, _LANE), jnp.float32),
            pltpu.VMEM((_LANE, _D), jnp.float32),
        ],
        compiler_params=pltpu.CompilerParams(
            dimension_semantics=("parallel",),
        ),
    )(inputs, Wr, br.reshape(1, _NUM_EXPERTS), We, be,
      Wq, bq.reshape(1, _D), Wk, bk.reshape(1, _D),
      Wv, bv.reshape(1, _D), Wo, bo.reshape(1, _D))
    return out


# speedup vs baseline: 1.0081x; 1.0026x over previous
import jax
import jax.numpy as jnp
from jax.experimental import pallas as pl
from jax.experimental.pallas import tpu as pltpu

_NUM_EXPERTS = 8
_D = 768
_HEADS = 12
_DEPTH = 64
_LANE = 128
_BT = 1024
_NEG = -1e30


def _fused_body(x_ref, wr_ref, br_ref, we_ref, be_ref, wq_ref, bq_ref,
                wk_ref, bk_ref, wv_ref, bv_ref, wo_ref, bo_ref, o_ref,
                wrp_ref, brp_ref, hmap_ref, hmapt_ref):
    wrp_ref[...] = jnp.concatenate(
        [wr_ref[...], jnp.zeros((_D, _LANE - _NUM_EXPERTS), jnp.float32)],
        axis=1)
    brp_ref[...] = jnp.concatenate(
        [br_ref[...], jnp.full((1, _LANE - _NUM_EXPERTS), _NEG,
                               jnp.float32)], axis=1)
    di = jax.lax.broadcasted_iota(jnp.int32, (_D, _LANE), 0) // _DEPTH
    hi = jax.lax.broadcasted_iota(jnp.int32, (_D, _LANE), 1)
    hmap_ref[...] = (di == hi).astype(jnp.float32)
    dit = jax.lax.broadcasted_iota(jnp.int32, (_LANE, _D), 1) // _DEPTH
    hit = jax.lax.broadcasted_iota(jnp.int32, (_LANE, _D), 0)
    hmapt_ref[...] = (dit == hit).astype(jnp.float32)

    x = x_ref[...]

    logits = jnp.dot(x, wrp_ref[...], preferred_element_type=jnp.float32)
    logits = logits + brp_ref[...]
    m = jnp.max(logits, axis=-1, keepdims=True)
    e = jnp.exp(logits - m)
    probs = e / jnp.sum(e, axis=-1, keepdims=True)

    cols = jax.lax.broadcasted_iota(jnp.int32, probs.shape, 1)
    p1 = jnp.max(probs, axis=-1, keepdims=True)
    i1 = jnp.min(jnp.where(probs >= p1, cols, _LANE), axis=-1, keepdims=True)
    probs_m = jnp.where(cols == i1, -1.0, probs)
    p2 = jnp.max(probs_m, axis=-1, keepdims=True)
    i2 = jnp.min(jnp.where(probs_m >= p2, cols, _LANE), axis=-1, keepdims=True)
    sel = (cols == i1) | (cols == i2)
    w = jnp.where(sel, probs, 0.0)

    combined = jnp.zeros((x.shape[0], _D), dtype=jnp.float32)
    for i_e in range(_NUM_EXPERTS):
        eo = jnp.dot(x, we_ref[i_e], preferred_element_type=jnp.float32)
        eo = jnp.maximum(eo + be_ref[i_e:i_e + 1, :], 0.0)
        combined = combined + eo * w[:, i_e:i_e + 1]

    q = jnp.dot(combined, wq_ref[...], preferred_element_type=jnp.float32) + bq_ref[...]
    k = jnp.dot(combined, wk_ref[...], preferred_element_type=jnp.float32) + bk_ref[...]
    v = jnp.dot(combined, wv_ref[...], preferred_element_type=jnp.float32) + bv_ref[...]
    s = jnp.dot(q * k, hmap_ref[...], preferred_element_type=jnp.float32)
    s = s * (1.0 / jnp.sqrt(jnp.float32(_DEPTH)))
    s = jnp.where(cols < _HEADS, s, _NEG)
    sm = jnp.max(s, axis=-1, keepdims=True)
    se = jnp.exp(s - sm)
    aw = se / jnp.sum(se, axis=-1, keepdims=True)
    wb = jnp.dot(aw, hmapt_ref[...], preferred_element_type=jnp.float32)
    out = jnp.dot(wb * v, wo_ref[...], preferred_element_type=jnp.float32)
    o_ref[...] = out + bo_ref[...]


@jax.jit
def kernel(inputs, Wr, br, We, be, Wq, bq, Wk, bk, Wv, bv, Wo, bo):
    n = inputs.shape[0]
    grid = (n // _BT,)
    full = lambda shape: pl.BlockSpec(shape, lambda i: (0,) * len(shape))
    out = pl.pallas_call(
        _fused_body,
        grid=grid,
        in_specs=[
            pl.BlockSpec((_BT, _D), lambda i: (i, 0)),
            full((_D, _NUM_EXPERTS)),
            full((1, _NUM_EXPERTS)),
            full((_NUM_EXPERTS, _D, _D)),
            full((_NUM_EXPERTS, _D)),
            full((_D, _D)), full((1, _D)),
            full((_D, _D)), full((1, _D)),
            full((_D, _D)), full((1, _D)),
            full((_D, _D)), full((1, _D)),
        ],
        out_specs=pl.BlockSpec((_BT, _D), lambda i: (i, 0)),
        out_shape=jax.ShapeDtypeStruct((n, _D), jnp.float32),
        scratch_shapes=[
            pltpu.VMEM((_D, _LANE), jnp.float32),
            pltpu.VMEM((1, _LANE), jnp.float32),
            pltpu.VMEM((_D, _LANE), jnp.float32),
            pltpu.VMEM((_LANE, _D), jnp.float32),
        ],
        compiler_params=pltpu.CompilerParams(
            dimension_semantics=("parallel",),
        ),
    )(inputs, Wr, br.reshape(1, _NUM_EXPERTS), We, be,
      Wq, bq.reshape(1, _D), Wk, bk.reshape(1, _D),
      Wv, bv.reshape(1, _D), Wo, bo.reshape(1, _D))
    return out
